# Initial kernel scaffold; baseline (speedup 1.0000x reference)
import functools

import jax
import jax.numpy as jnp
from jax import lax
from jax.experimental import pallas as pl
from jax.experimental.pallas import tpu as pltpu
from jax.experimental.pallas import tpu_sc as plsc

B = 4096
T = 48
EMB = 32
H = 1024
N_TS_CONT = 8
N_TS_CAT = 4
N_ST_CONT = 10
N_ST_CAT = 6
VOCAB = 1000
EPS = 1e-5

K_CAT = N_TS_CAT * T + N_ST_CAT
R_TOTAL = B * K_CAT
D_CAT = K_CAT * EMB
KC = T * N_TS_CONT + N_ST_CONT
KCP = 512

NW = 32
R_W = R_TOTAL // NW
CHUNK = 2304
N_CHUNK = R_W // CHUNK
G = 128
N_G = CHUNK // G

BM = 512
NB = B // BM


def _gather_body(tab_hbm, idx_hbm, out_hbm, idx_v, rows_v, sem):
    wid = lax.axis_index("s") * 2 + lax.axis_index("c")

    def chunk(c, carry):
        row0 = wid * R_W + c * CHUNK
        g0 = wid * (R_W // G) + c * N_G
        pltpu.sync_copy(idx_hbm.at[pl.ds(g0, N_G)], idx_v)
        descs = [
            pltpu.async_copy(
                tab_hbm.at[idx_v.at[j]],
                rows_v.at[pl.ds(j * G, G)],
                sem,
            )
            for j in range(N_G)
        ]
        for d in descs:
            d.wait()
        pltpu.sync_copy(rows_v, out_hbm.at[pl.ds(row0, CHUNK)])
        return carry

    lax.fori_loop(0, N_CHUNK, chunk, 0)


@functools.lru_cache(maxsize=None)
def _gather_fn():
    return pl.kernel(
        _gather_body,
        out_type=jax.ShapeDtypeStruct((R_TOTAL, EMB), jnp.float32),
        mesh=plsc.VectorSubcoreMesh(core_axis_name="c", subcore_axis_name="s"),
        scratch_types=[
            pltpu.VMEM((N_G, G), jnp.int32),
            pltpu.VMEM((CHUNK, EMB), jnp.float32),
            pltpu.SemaphoreType.DMA,
        ],
    )


def _mm_body(x_ref, xc_ref, w_ref, wc_ref, b_ref, o_ref):
    acc = jnp.dot(x_ref[...], w_ref[...], preferred_element_type=jnp.float32)
    acc += jnp.dot(xc_ref[...], wc_ref[...], preferred_element_type=jnp.float32)
    o_ref[...] = acc + b_ref[...]


def _chain_body(h0_ref, w_ref, bh_ref, g_ref, be_ref, o_ref, hs):
    i = pl.program_id(0)

    @pl.when(i == 0)
    def _():
        hs[...] = h0_ref[...]

    r = jnp.maximum(hs[...], 0.0)
    mu = jnp.mean(r, axis=0, keepdims=True)
    d = r - mu
    var = jnp.mean(d * d, axis=0, keepdims=True)
    hn = d * lax.rsqrt(var + EPS) * g_ref[...] + be_ref[...]
    h2 = jnp.dot(hn, w_ref[0], preferred_element_type=jnp.float32) + bh_ref[...]

    @pl.when(i < 2)
    def _():
        hs[...] = h2

    @pl.when(i == 2)
    def _():
        o_ref[...] = h2


def _mlp(x_cat, xc, W0cat, W0c, b0r, Wh, bh, gamma, beta):
    h0 = pl.pallas_call(
        _mm_body,
        grid=(NB,),
        in_specs=[
            pl.BlockSpec((BM, D_CAT), lambda i: (i, 0)),
            pl.BlockSpec((BM, KCP), lambda i: (i, 0)),
            pl.BlockSpec((D_CAT, H), lambda i: (0, 0)),
            pl.BlockSpec((KCP, H), lambda i: (0, 0)),
            pl.BlockSpec((1, H), lambda i: (0, 0)),
        ],
        out_specs=pl.BlockSpec((BM, H), lambda i: (i, 0)),
        out_shape=jax.ShapeDtypeStruct((B, H), jnp.float32),
        compiler_params=pltpu.CompilerParams(
            dimension_semantics=("arbitrary",),
        ),
    )(x_cat, xc, W0cat, W0c, b0r)

    out = pl.pallas_call(
        _chain_body,
        grid=(3,),
        in_specs=[
            pl.BlockSpec((B, H), lambda l: (0, 0)),
            pl.BlockSpec((1, H, H), lambda l: (l, 0, 0)),
            pl.BlockSpec((1, H), lambda l: (l, 0)),
            pl.BlockSpec((1, H), lambda l: (l, 0)),
            pl.BlockSpec((1, H), lambda l: (l, 0)),
        ],
        out_specs=pl.BlockSpec((B, H), lambda l: (0, 0)),
        out_shape=jax.ShapeDtypeStruct((B, H), jnp.float32),
        scratch_shapes=[pltpu.VMEM((B, H), jnp.float32)],
        compiler_params=pltpu.CompilerParams(
            dimension_semantics=("arbitrary",),
        ),
    )(h0, Wh, bh, gamma, beta)
    return out


def kernel(ts_cont_feats, ts_cat_feats, static_cont_feats, static_cat_feats,
           ts_tables, static_tables, W0, b0, Wh, bh, gamma, beta):
    idx_ts = ts_cat_feats.astype(jnp.int32).transpose(0, 2, 1) \
        + (jnp.arange(N_TS_CAT, dtype=jnp.int32) * VOCAB)[None, :, None]
    idx_st = static_cat_feats.astype(jnp.int32) \
        + N_TS_CAT * VOCAB + jnp.arange(N_ST_CAT, dtype=jnp.int32) * VOCAB
    idx = jnp.concatenate([idx_ts.reshape(B, N_TS_CAT * T), idx_st], axis=1)
    idx2d = idx.reshape(R_TOTAL // G, G)
    tab = jnp.concatenate(
        [ts_tables.reshape(N_TS_CAT * VOCAB, EMB),
         static_tables.reshape(N_ST_CAT * VOCAB, EMB)], axis=0)

    x_cat = _gather_fn()(tab, idx2d).reshape(B, D_CAT)

    xc = jnp.concatenate(
        [ts_cont_feats.astype(jnp.float32).reshape(B, T * N_TS_CONT),
         static_cont_feats.astype(jnp.float32)], axis=1)
    xc = jnp.pad(xc, ((0, 0), (0, KCP - KC)))
    W0c = jnp.pad(W0[D_CAT:], ((0, KCP - KC), (0, 0)))

    return _mlp(x_cat, xc, W0[:D_CAT], W0c, b0.reshape(1, H),
                Wh, bh, gamma, beta)


# speedup vs baseline: 6.5694x; 6.5694x over previous
import functools

import jax
import jax.numpy as jnp
from jax import lax
from jax.experimental import pallas as pl
from jax.experimental.pallas import tpu as pltpu
from jax.experimental.pallas import tpu_sc as plsc

B = 4096
T = 48
EMB = 32
H = 1024
N_TS_CONT = 8
N_TS_CAT = 4
N_ST_CONT = 10
N_ST_CAT = 6
VOCAB = 1000
EPS = 1e-5

K_CAT = N_TS_CAT * T + N_ST_CAT
R_TOTAL = B * K_CAT
D_CAT = K_CAT * EMB
KC = T * N_TS_CONT + N_ST_CONT
KCP = 512

NW = 32
R_W = R_TOTAL // NW
CHUNK = 2304
N_CHUNK = R_W // CHUNK
G = 128
N_G = CHUNK // G

BM = 512
NB = B // BM


def _gather_body(tab_hbm, idx_hbm, out_hbm, idx_v, rows_v, sem):
    wid = lax.axis_index("s") * 2 + lax.axis_index("c")
    pltpu.sync_copy(idx_hbm.at[wid], idx_v)

    def chunk(c, carry):
        row0 = wid * R_W + c * CHUNK
        descs = [
            pltpu.async_copy(
                tab_hbm.at[idx_v.at[c * N_G + j]],
                rows_v.at[pl.ds(j * G, G)],
                sem,
            )
            for j in range(N_G)
        ]
        for d in descs:
            d.wait()
        pltpu.sync_copy(rows_v, out_hbm.at[pl.ds(row0, CHUNK)])
        return carry

    lax.fori_loop(0, N_CHUNK, chunk, 0)


@functools.lru_cache(maxsize=None)
def _gather_fn():
    return pl.kernel(
        _gather_body,
        out_type=jax.ShapeDtypeStruct((R_TOTAL, EMB), jnp.float32),
        mesh=plsc.VectorSubcoreMesh(core_axis_name="c", subcore_axis_name="s"),
        scratch_types=[
            pltpu.VMEM((R_W // G, G), jnp.int32),
            pltpu.VMEM((CHUNK, EMB), jnp.float32),
            pltpu.SemaphoreType.DMA,
        ],
        compiler_params=pltpu.CompilerParams(use_tc_tiling_on_sc=False),
    )


def _mm_body(x_ref, xc_ref, w_ref, wc_ref, b_ref, o_ref):
    acc = jnp.dot(x_ref[...], w_ref[...], preferred_element_type=jnp.float32)
    acc += jnp.dot(xc_ref[...], wc_ref[...], preferred_element_type=jnp.float32)
    o_ref[...] = acc + b_ref[...]


def _chain_body(h0_ref, w_ref, bh_ref, g_ref, be_ref, o_ref, hs, stats):
    l = pl.program_id(0)
    j = pl.program_id(1)

    @pl.when((l == 0) & (j > 0))
    def _():
        hs[pl.ds((j - 1) * BM, BM), :] = h0_ref[...]

    @pl.when((l > 0) & (j == 0))
    def _():
        def s_body(k, acc):
            r = jnp.maximum(hs[pl.ds(k * BM, BM), :], 0.0)
            return acc + jnp.sum(r, axis=0, keepdims=True)
        mu = lax.fori_loop(0, NB, s_body, jnp.zeros((1, H), jnp.float32)) / B

        def v_body(k, acc):
            r = jnp.maximum(hs[pl.ds(k * BM, BM), :], 0.0)
            d = r - mu
            return acc + jnp.sum(d * d, axis=0, keepdims=True)
        var = lax.fori_loop(0, NB, v_body, jnp.zeros((1, H), jnp.float32)) / B
        stats[0:1, :] = mu
        stats[1:2, :] = lax.rsqrt(var + EPS)

    @pl.when((l > 0) & (j > 0))
    def _():
        b = (j - 1) * BM
        r = jnp.maximum(hs[pl.ds(b, BM), :], 0.0)
        hn = (r - stats[0:1, :]) * stats[1:2, :] * g_ref[0] + be_ref[0]
        h2 = jnp.dot(hn, w_ref[0], preferred_element_type=jnp.float32) \
            + bh_ref[0]

        @pl.when(l < 3)
        def _():
            hs[pl.ds(b, BM), :] = h2

        @pl.when(l == 3)
        def _():
            o_ref[...] = h2


def _mlp(x_cat, xc, W0cat, W0c, b0r, Wh, bh, gamma, beta):
    h0 = pl.pallas_call(
        _mm_body,
        grid=(NB,),
        in_specs=[
            pl.BlockSpec((BM, D_CAT), lambda i: (i, 0)),
            pl.BlockSpec((BM, KCP), lambda i: (i, 0)),
            pl.BlockSpec((D_CAT, H), lambda i: (0, 0)),
            pl.BlockSpec((KCP, H), lambda i: (0, 0)),
            pl.BlockSpec((1, H), lambda i: (0, 0)),
        ],
        out_specs=pl.BlockSpec((BM, H), lambda i: (i, 0)),
        out_shape=jax.ShapeDtypeStruct((B, H), jnp.float32),
        compiler_params=pltpu.CompilerParams(
            dimension_semantics=("arbitrary",),
        ),
    )(x_cat, xc, W0cat, W0c, b0r)

    blk = lambda l, j: (jnp.maximum(j, 1) - 1, 0)
    lyr = lambda l, j: (jnp.maximum(l, 1) - 1, 0, 0)
    out = pl.pallas_call(
        _chain_body,
        grid=(4, NB + 1),
        in_specs=[
            pl.BlockSpec((BM, H), blk),
            pl.BlockSpec((1, H, H), lyr),
            pl.BlockSpec((1, 1, H), lyr),
            pl.BlockSpec((1, 1, H), lyr),
            pl.BlockSpec((1, 1, H), lyr),
        ],
        out_specs=pl.BlockSpec((BM, H), blk),
        out_shape=jax.ShapeDtypeStruct((B, H), jnp.float32),
        scratch_shapes=[
            pltpu.VMEM((B, H), jnp.float32),
            pltpu.VMEM((8, H), jnp.float32),
        ],
        compiler_params=pltpu.CompilerParams(
            dimension_semantics=("arbitrary", "arbitrary"),
        ),
    )(h0, Wh, bh.reshape(3, 1, H), gamma.reshape(3, 1, H),
      beta.reshape(3, 1, H))
    return out


def kernel(ts_cont_feats, ts_cat_feats, static_cont_feats, static_cat_feats,
           ts_tables, static_tables, W0, b0, Wh, bh, gamma, beta):
    idx_ts = ts_cat_feats.astype(jnp.int32).transpose(0, 2, 1) \
        + (jnp.arange(N_TS_CAT, dtype=jnp.int32) * VOCAB)[None, :, None]
    idx_st = static_cat_feats.astype(jnp.int32) \
        + N_TS_CAT * VOCAB + jnp.arange(N_ST_CAT, dtype=jnp.int32) * VOCAB
    idx = jnp.concatenate([idx_ts.reshape(B, N_TS_CAT * T), idx_st], axis=1)
    idx3d = idx.reshape(NW, R_W // G, G)
    tab = jnp.concatenate(
        [ts_tables.reshape(N_TS_CAT * VOCAB, EMB),
         static_tables.reshape(N_ST_CAT * VOCAB, EMB)], axis=0)

    x_cat = _gather_fn()(tab, idx3d).reshape(B, D_CAT)

    xc = jnp.concatenate(
        [ts_cont_feats.astype(jnp.float32).reshape(B, T * N_TS_CONT),
         static_cont_feats.astype(jnp.float32)], axis=1)
    xc = jnp.pad(xc, ((0, 0), (0, KCP - KC)))
    W0c = jnp.pad(W0[D_CAT:], ((0, KCP - KC), (0, 0)))

    return _mlp(x_cat, xc, W0[:D_CAT], W0c, b0.reshape(1, H),
                Wh, bh, gamma, beta)


# speedup vs baseline: 6.8495x; 1.0426x over previous
import functools

import jax
import jax.numpy as jnp
from jax import lax
from jax.experimental import pallas as pl
from jax.experimental.pallas import tpu as pltpu
from jax.experimental.pallas import tpu_sc as plsc

B = 4096
T = 48
EMB = 32
H = 1024
N_TS_CONT = 8
N_TS_CAT = 4
N_ST_CONT = 10
N_ST_CAT = 6
VOCAB = 1000
EPS = 1e-5

K_CAT = N_TS_CAT * T + N_ST_CAT
R_TOTAL = B * K_CAT
D_CAT = K_CAT * EMB
KC = T * N_TS_CONT + N_ST_CONT
KCP = 512

NW = 32
R_W = R_TOTAL // NW
CHUNK = 2304
N_CHUNK = R_W // CHUNK
G = 128
N_G = CHUNK // G

BM = 512
NB = B // BM


def _gather_body(tab_hbm, idx_hbm, out_hbm, idx_v, rows_v, sem):
    wid = lax.axis_index("s") * 2 + lax.axis_index("c")
    pltpu.sync_copy(idx_hbm.at[wid], idx_v)

    def chunk(c, carry):
        row0 = wid * R_W + c * CHUNK
        descs = [
            pltpu.async_copy(
                tab_hbm.at[idx_v.at[c * N_G + j]],
                rows_v.at[pl.ds(j * G, G)],
                sem,
            )
            for j in range(N_G)
        ]
        for d in descs:
            d.wait()
        pltpu.sync_copy(rows_v, out_hbm.at[pl.ds(row0, CHUNK)])
        return carry

    lax.fori_loop(0, N_CHUNK, chunk, 0)


@functools.lru_cache(maxsize=None)
def _gather_fn():
    return pl.kernel(
        _gather_body,
        out_type=jax.ShapeDtypeStruct((R_TOTAL, EMB), jnp.float32),
        mesh=plsc.VectorSubcoreMesh(core_axis_name="c", subcore_axis_name="s"),
        scratch_types=[
            pltpu.VMEM((R_W // G, G), jnp.int32),
            pltpu.VMEM((CHUNK, EMB), jnp.float32),
            pltpu.SemaphoreType.DMA,
        ],
        compiler_params=pltpu.CompilerParams(use_tc_tiling_on_sc=False),
    )


def _mm_body(x_ref, xc_ref, w_ref, wc_ref, b_ref, o_ref):
    acc = jnp.dot(x_ref[...].astype(jnp.bfloat16), w_ref[...],
                  preferred_element_type=jnp.float32)
    acc += jnp.dot(xc_ref[...], wc_ref[...],
                   preferred_element_type=jnp.float32)
    o_ref[...] = acc + b_ref[...]


def _chain_body(h0_ref, w_ref, bh_ref, g_ref, be_ref, o_ref, hs, stats):
    l = pl.program_id(0)
    j = pl.program_id(1)

    @pl.when((l == 0) & (j > 0))
    def _():
        hs[pl.ds((j - 1) * BM, BM), :] = h0_ref[...]

    @pl.when((l > 0) & (j == 0))
    def _():
        def s_body(k, acc):
            r = jnp.maximum(hs[pl.ds(k * BM, BM), :], 0.0)
            return acc + jnp.sum(r, axis=0, keepdims=True)
        mu = lax.fori_loop(0, NB, s_body, jnp.zeros((1, H), jnp.float32)) / B

        def v_body(k, acc):
            r = jnp.maximum(hs[pl.ds(k * BM, BM), :], 0.0)
            d = r - mu
            return acc + jnp.sum(d * d, axis=0, keepdims=True)
        var = lax.fori_loop(0, NB, v_body, jnp.zeros((1, H), jnp.float32)) / B
        stats[0:1, :] = mu
        stats[1:2, :] = lax.rsqrt(var + EPS)

    @pl.when((l > 0) & (j > 0))
    def _():
        b = (j - 1) * BM
        r = jnp.maximum(hs[pl.ds(b, BM), :], 0.0)
        hn = (r - stats[0:1, :]) * stats[1:2, :] * g_ref[0] + be_ref[0]
        h2 = jnp.dot(hn.astype(jnp.bfloat16), w_ref[0],
                     preferred_element_type=jnp.float32) + bh_ref[0]

        @pl.when(l < 3)
        def _():
            hs[pl.ds(b, BM), :] = h2

        @pl.when(l == 3)
        def _():
            o_ref[...] = h2


def _mlp(x_cat, xc, W0cat, W0c, b0r, Wh, bh, gamma, beta):
    h0 = pl.pallas_call(
        _mm_body,
        grid=(NB,),
        in_specs=[
            pl.BlockSpec((BM, D_CAT), lambda i: (i, 0)),
            pl.BlockSpec((BM, KCP), lambda i: (i, 0)),
            pl.BlockSpec((D_CAT, H), lambda i: (0, 0)),
            pl.BlockSpec((KCP, H), lambda i: (0, 0)),
            pl.BlockSpec((1, H), lambda i: (0, 0)),
        ],
        out_specs=pl.BlockSpec((BM, H), lambda i: (i, 0)),
        out_shape=jax.ShapeDtypeStruct((B, H), jnp.float32),
        compiler_params=pltpu.CompilerParams(
            dimension_semantics=("arbitrary",),
        ),
    )(x_cat, xc, W0cat, W0c, b0r)

    blk = lambda l, j: (jnp.maximum(j, 1) - 1, 0)
    lyr = lambda l, j: (jnp.maximum(l, 1) - 1, 0, 0)
    out = pl.pallas_call(
        _chain_body,
        grid=(4, NB + 1),
        in_specs=[
            pl.BlockSpec((BM, H), blk),
            pl.BlockSpec((1, H, H), lyr),
            pl.BlockSpec((1, 1, H), lyr),
            pl.BlockSpec((1, 1, H), lyr),
            pl.BlockSpec((1, 1, H), lyr),
        ],
        out_specs=pl.BlockSpec((BM, H), blk),
        out_shape=jax.ShapeDtypeStruct((B, H), jnp.float32),
        scratch_shapes=[
            pltpu.VMEM((B, H), jnp.float32),
            pltpu.VMEM((8, H), jnp.float32),
        ],
        compiler_params=pltpu.CompilerParams(
            dimension_semantics=("arbitrary", "arbitrary"),
        ),
    )(h0, Wh, bh.reshape(3, 1, H), gamma.reshape(3, 1, H),
      beta.reshape(3, 1, H))
    return out


def kernel(ts_cont_feats, ts_cat_feats, static_cont_feats, static_cat_feats,
           ts_tables, static_tables, W0, b0, Wh, bh, gamma, beta):
    idx_ts = ts_cat_feats.astype(jnp.int32).transpose(0, 2, 1) \
        + (jnp.arange(N_TS_CAT, dtype=jnp.int32) * VOCAB)[None, :, None]
    idx_st = static_cat_feats.astype(jnp.int32) \
        + N_TS_CAT * VOCAB + jnp.arange(N_ST_CAT, dtype=jnp.int32) * VOCAB
    idx = jnp.concatenate([idx_ts.reshape(B, N_TS_CAT * T), idx_st], axis=1)
    idx3d = idx.reshape(NW, R_W // G, G)
    tab = jnp.concatenate(
        [ts_tables.reshape(N_TS_CAT * VOCAB, EMB),
         static_tables.reshape(N_ST_CAT * VOCAB, EMB)], axis=0)

    x_cat = _gather_fn()(tab, idx3d).reshape(B, D_CAT)

    xc = jnp.concatenate(
        [ts_cont_feats.astype(jnp.float32).reshape(B, T * N_TS_CONT),
         static_cont_feats.astype(jnp.float32)], axis=1)
    xc = jnp.pad(xc, ((0, 0), (0, KCP - KC))).astype(jnp.bfloat16)
    W0c = jnp.pad(W0[D_CAT:], ((0, KCP - KC), (0, 0))).astype(jnp.bfloat16)

    return _mlp(x_cat, xc, W0[:D_CAT].astype(jnp.bfloat16), W0c,
                b0.reshape(1, H), Wh.astype(jnp.bfloat16), bh, gamma, beta)


# speedup vs baseline: 7.0155x; 1.0242x over previous
import functools

import jax
import jax.numpy as jnp
from jax import lax
from jax.experimental import pallas as pl
from jax.experimental.pallas import tpu as pltpu
from jax.experimental.pallas import tpu_sc as plsc

B = 4096
T = 48
EMB = 32
H = 1024
N_TS_CONT = 8
N_TS_CAT = 4
N_ST_CONT = 10
N_ST_CAT = 6
VOCAB = 1000
EPS = 1e-5

K_CAT = N_TS_CAT * T + N_ST_CAT
D_CAT = K_CAT * EMB
KC = T * N_TS_CONT + N_ST_CONT
KCP = 512

NW = 32
G = 128

BM = 512
NB = B // BM
NH = 2
BH = B // NH
NBH = BH // BM


def _make_gather_body(r_w, n_g, n_chunk):
    chunk_rows = n_g * G

    def body(tab_hbm, idx_hbm, out_hbm, idx_v, rows_v, sem):
        wid = lax.axis_index("s") * 2 + lax.axis_index("c")
        pltpu.sync_copy(idx_hbm.at[wid], idx_v)

        def chunk(c, carry):
            row0 = wid * r_w + c * chunk_rows
            descs = [
                pltpu.async_copy(
                    tab_hbm.at[idx_v.at[c * n_g + j]],
                    rows_v.at[pl.ds(j * G, G)],
                    sem,
                )
                for j in range(n_g)
            ]
            for d in descs:
                d.wait()
            pltpu.sync_copy(rows_v, out_hbm.at[pl.ds(row0, chunk_rows)])
            return carry

        lax.fori_loop(0, n_chunk, chunk, 0)

    return body


@functools.lru_cache(maxsize=None)
def _gather_fn(r_total, n_g, n_chunk):
    r_w = r_total // NW
    return pl.kernel(
        _make_gather_body(r_w, n_g, n_chunk),
        out_type=jax.ShapeDtypeStruct((r_total, EMB), jnp.float32),
        mesh=plsc.VectorSubcoreMesh(core_axis_name="c", subcore_axis_name="s"),
        scratch_types=[
            pltpu.VMEM((r_w // G, G), jnp.int32),
            pltpu.VMEM((n_g * G, EMB), jnp.float32),
            pltpu.SemaphoreType.DMA,
        ],
        compiler_params=pltpu.CompilerParams(use_tc_tiling_on_sc=False),
    )


def _mm_body(x_ref, xc_ref, w_ref, wc_ref, b_ref, o_ref):
    acc = jnp.dot(x_ref[...].astype(jnp.bfloat16), w_ref[...],
                  preferred_element_type=jnp.float32)
    acc += jnp.dot(xc_ref[...], wc_ref[...],
                   preferred_element_type=jnp.float32)
    o_ref[...] = acc + b_ref[...]


def _chain_body(h0_ref, w_ref, bh_ref, g_ref, be_ref, o_ref, hs, stats, acc):
    l = pl.program_id(0)
    j = pl.program_id(1)

    def _accumulate(r):
        acc[0:1, :] += jnp.sum(r, axis=0, keepdims=True)
        acc[1:2, :] += jnp.sum(r * r, axis=0, keepdims=True)

    @pl.when((l == 0) & (j == 0))
    def _():
        acc[...] = jnp.zeros_like(acc)

    @pl.when((l > 0) & (j == 0))
    def _():
        mu = acc[0:1, :] * (1.0 / B)
        var = acc[1:2, :] * (1.0 / B) - mu * mu
        stats[0:1, :] = mu
        stats[1:2, :] = lax.rsqrt(var + EPS)
        acc[...] = jnp.zeros_like(acc)

    @pl.when((l == 0) & (j > 0))
    def _():
        blk = h0_ref[...]
        hs[pl.ds((j - 1) * BM, BM), :] = blk
        _accumulate(jnp.maximum(blk, 0.0))

    @pl.when((l > 0) & (j > 0))
    def _():
        b = (j - 1) * BM
        r = jnp.maximum(hs[pl.ds(b, BM), :], 0.0)
        hn = (r - stats[0:1, :]) * stats[1:2, :] * g_ref[0] + be_ref[0]
        h2 = jnp.dot(hn.astype(jnp.bfloat16), w_ref[0],
                     preferred_element_type=jnp.float32) + bh_ref[0]

        @pl.when(l < 3)
        def _():
            hs[pl.ds(b, BM), :] = h2
            _accumulate(jnp.maximum(h2, 0.0))

        @pl.when(l == 3)
        def _():
            o_ref[...] = h2


def _first_layer(x_cat, xc, W0cat, W0c, b0r):
    return pl.pallas_call(
        _mm_body,
        grid=(NBH,),
        in_specs=[
            pl.BlockSpec((BM, D_CAT), lambda i: (i, 0)),
            pl.BlockSpec((BM, KCP), lambda i: (i, 0)),
            pl.BlockSpec((D_CAT, H), lambda i: (0, 0)),
            pl.BlockSpec((KCP, H), lambda i: (0, 0)),
            pl.BlockSpec((1, H), lambda i: (0, 0)),
        ],
        out_specs=pl.BlockSpec((BM, H), lambda i: (i, 0)),
        out_shape=jax.ShapeDtypeStruct((BH, H), jnp.float32),
        compiler_params=pltpu.CompilerParams(
            dimension_semantics=("arbitrary",),
        ),
    )(x_cat, xc, W0cat, W0c, b0r)


def _chain(h0, Wh, bh, gamma, beta):
    blk = lambda l, j: (jnp.maximum(j, 1) - 1, 0)
    lyr = lambda l, j: (jnp.maximum(l, 1) - 1, 0, 0)
    return pl.pallas_call(
        _chain_body,
        grid=(4, NB + 1),
        in_specs=[
            pl.BlockSpec((BM, H), blk),
            pl.BlockSpec((1, H, H), lyr),
            pl.BlockSpec((1, 1, H), lyr),
            pl.BlockSpec((1, 1, H), lyr),
            pl.BlockSpec((1, 1, H), lyr),
        ],
        out_specs=pl.BlockSpec((BM, H), blk),
        out_shape=jax.ShapeDtypeStruct((B, H), jnp.float32),
        scratch_shapes=[
            pltpu.VMEM((B, H), jnp.float32),
            pltpu.VMEM((8, H), jnp.float32),
            pltpu.VMEM((8, H), jnp.float32),
        ],
        compiler_params=pltpu.CompilerParams(
            dimension_semantics=("arbitrary", "arbitrary"),
        ),
    )(h0, Wh, bh.reshape(3, 1, H), gamma.reshape(3, 1, H),
      beta.reshape(3, 1, H))


def kernel(ts_cont_feats, ts_cat_feats, static_cont_feats, static_cat_feats,
           ts_tables, static_tables, W0, b0, Wh, bh, gamma, beta):
    idx_ts = ts_cat_feats.astype(jnp.int32).transpose(0, 2, 1) \
        + (jnp.arange(N_TS_CAT, dtype=jnp.int32) * VOCAB)[None, :, None]
    idx_st = static_cat_feats.astype(jnp.int32) \
        + N_TS_CAT * VOCAB + jnp.arange(N_ST_CAT, dtype=jnp.int32) * VOCAB
    idx = jnp.concatenate([idx_ts.reshape(B, N_TS_CAT * T), idx_st], axis=1)
    tab = jnp.concatenate(
        [ts_tables.reshape(N_TS_CAT * VOCAB, EMB),
         static_tables.reshape(N_ST_CAT * VOCAB, EMB)], axis=0)

    xc = jnp.concatenate(
        [ts_cont_feats.astype(jnp.float32).reshape(B, T * N_TS_CONT),
         static_cont_feats.astype(jnp.float32)], axis=1)
    xc = jnp.pad(xc, ((0, 0), (0, KCP - KC))).astype(jnp.bfloat16)
    W0cat = W0[:D_CAT].astype(jnp.bfloat16)
    W0c = jnp.pad(W0[D_CAT:], ((0, KCP - KC), (0, 0))).astype(jnp.bfloat16)
    b0r = b0.reshape(1, H)

    r_half = BH * K_CAT
    gather = _gather_fn(r_half, 9, 11)
    halves = []
    for hlf in range(NH):
        idx_h = idx[hlf * BH:(hlf + 1) * BH].reshape(NW, r_half // (NW * G), G)
        x_cat = gather(tab, idx_h).reshape(BH, D_CAT)
        halves.append(_first_layer(
            x_cat, xc[hlf * BH:(hlf + 1) * BH], W0cat, W0c, b0r))

    h0 = jnp.concatenate(halves, axis=0)
    return _chain(h0, Wh.astype(jnp.bfloat16), bh, gamma, beta)
